# baseline (speedup 1.0000x reference)
import functools

import jax
import jax.numpy as jnp
from jax import lax
from jax.experimental import pallas as pl
from jax.experimental.pallas import tpu as pltpu
from jax.experimental.pallas import tpu_sc as plsc

C = 1000
N = 1024 * 50

_info = plsc.get_sparse_core_info()
_NC, _NS, _L = _info.num_cores, _info.num_subcores, _info.num_lanes
_NW = _NC * _NS
_CB = 8
_NCB = C // _CB
_ICHUNK = 2048
_NIC = N // _ICHUNK


def _lse_t_body(tab_ref, tabt_ref, lse_ref):
    x = tab_ref[...]
    m = jnp.max(x, axis=1)
    s = jnp.sum(jnp.exp(x - m[:, None]), axis=1)
    lse_ref[...] = m + jnp.log(s)
    tabt_ref[...] = x.T


def _lse_and_transpose(table):
    return pl.pallas_call(
        _lse_t_body,
        out_shape=[
            jax.ShapeDtypeStruct((C, C), jnp.float32),
            jax.ShapeDtypeStruct((C,), jnp.float32),
        ],
    )(table)


_sc_mesh = plsc.VectorSubcoreMesh(core_axis_name="c", subcore_axis_name="s")


@functools.partial(
    pl.kernel,
    mesh=_sc_mesh,
    out_type=[
        jax.ShapeDtypeStruct((C, N), jnp.float32),
        jax.ShapeDtypeStruct((_NW, _L), jnp.float32),
    ],
    scratch_types=[
        pltpu.VMEM((_CB, C), jnp.float32),
        pltpu.VMEM((_ICHUNK,), jnp.int32),
        pltpu.VMEM((_ICHUNK,), jnp.int32),
        pltpu.VMEM((_CB, _ICHUNK), jnp.float32),
        pltpu.VMEM((C,), jnp.float32),
        pltpu.VMEM((_L,), jnp.float32),
        pltpu.SemaphoreType.DMA,
    ],
    compiler_params=pltpu.CompilerParams(needs_layout_passes=False),
)
def _sc_gather_t(tabt_hbm, idx_hbm, tgt_hbm, lse_hbm, out_hbm, part_hbm,
                 tbl_v, idx_v, tgt_v, out_v, lse_v, acc_v, sem):
    wid = lax.axis_index("s") * _NC + lax.axis_index("c")
    pltpu.sync_copy(lse_hbm, lse_v)
    acc_v[...] = jnp.zeros((_L,), jnp.float32)

    def cblock_body(g):
        c0 = g * _CB
        pltpu.sync_copy(tabt_hbm.at[pl.ds(c0, _CB)], tbl_v)

        def ichunk_body(ic, _):
            i0 = ic * _ICHUNK
            pltpu.sync_copy(idx_hbm.at[pl.ds(i0, _ICHUNK)], idx_v)
            pltpu.sync_copy(tgt_hbm.at[pl.ds(i0, _ICHUNK)], tgt_v)

            def igroup_body(v, acc):
                idx16 = idx_v[pl.ds(v * _L, _L)]
                tgt16 = tgt_v[pl.ds(v * _L, _L)]
                for c in range(_CB):
                    row = plsc.load_gather(tbl_v,
                                           [jnp.full((_L,), c, jnp.int32),
                                            idx16])
                    out_v[c, pl.ds(v * _L, _L)] = row
                tloc = tgt16 - c0
                inblk = (tloc >= 0) & (tloc < _CB)
                jc = jnp.clip(tloc, 0, _CB - 1)
                tval = plsc.load_gather(tbl_v, [jc, idx16])
                lse_g = plsc.load_gather(lse_v, [idx16])
                return acc + jnp.where(inblk, lse_g - tval,
                                       jnp.zeros((_L,), jnp.float32))

            acc = lax.fori_loop(0, _ICHUNK // _L, igroup_body, acc_v[...])
            acc_v[...] = acc
            pltpu.sync_copy(out_v,
                            out_hbm.at[pl.ds(c0, _CB), pl.ds(i0, _ICHUNK)])
            return 0

        lax.fori_loop(0, _NIC, ichunk_body, 0)
        return g + _NW

    lax.while_loop(lambda g: g < _NCB, cblock_body, wid)
    pltpu.sync_copy(acc_v, part_hbm.at[wid])


def _loss_body(p_ref, out_ref):
    out_ref[...] = (jnp.sum(p_ref[...]) / N)[None]


def _loss_reduce(partials):
    return pl.pallas_call(
        _loss_body,
        out_shape=jax.ShapeDtypeStruct((1,), jnp.float32),
    )(partials)


def kernel(table, idx, targets):
    tableT, lse = _lse_and_transpose(table)
    idx_f = idx.reshape(-1)
    tgt_f = targets.reshape(-1)
    logitsT, partials = _sc_gather_t(tableT, idx_f, tgt_f, lse)
    loss = _loss_reduce(partials)[0]
    return logitsT.T, loss


# speedup vs baseline: 1.1128x; 1.1128x over previous
import functools

import jax
import jax.numpy as jnp
from jax import lax
from jax.experimental import pallas as pl
from jax.experimental.pallas import tpu as pltpu
from jax.experimental.pallas import tpu_sc as plsc

C = 1000
N = 1024 * 50

_info = plsc.get_sparse_core_info()
_NC, _NS, _L = _info.num_cores, _info.num_subcores, _info.num_lanes
_NW = _NC * _NS
_CB = 8
_NCB = C // _CB
_ICHUNK = 2560
_NIC = N // _ICHUNK
_UNROLL = 8


def _lse_t_body(tab_ref, tabt_ref, lse_ref):
    x = tab_ref[...]
    m = jnp.max(x, axis=1)
    s = jnp.sum(jnp.exp(x - m[:, None]), axis=1)
    lse_ref[...] = m + jnp.log(s)
    tabt_ref[...] = x.T


def _lse_and_transpose(table):
    return pl.pallas_call(
        _lse_t_body,
        out_shape=[
            jax.ShapeDtypeStruct((C, C), jnp.float32),
            jax.ShapeDtypeStruct((C,), jnp.float32),
        ],
    )(table)


_sc_mesh = plsc.VectorSubcoreMesh(core_axis_name="c", subcore_axis_name="s")


@functools.partial(
    pl.kernel,
    mesh=_sc_mesh,
    out_type=[
        jax.ShapeDtypeStruct((C, N), jnp.float32),
        jax.ShapeDtypeStruct((_NW, _L), jnp.float32),
    ],
    scratch_types=[
        pltpu.VMEM((_CB, C), jnp.float32),
        pltpu.VMEM((_ICHUNK,), jnp.int32),
        pltpu.VMEM((_ICHUNK,), jnp.int32),
        pltpu.VMEM((2, _CB, _ICHUNK), jnp.float32),
        pltpu.VMEM((C,), jnp.float32),
        pltpu.VMEM((_L,), jnp.float32),
        pltpu.SemaphoreType.DMA,
        pltpu.SemaphoreType.DMA,
    ],
    compiler_params=pltpu.CompilerParams(needs_layout_passes=False),
)
def _sc_gather_t(tabt_hbm, idx_hbm, tgt_hbm, lse_hbm, out_hbm, part_hbm,
                 tbl_v, idx_v, tgt_v, out_v, lse_v, acc_v, sem0, sem1):
    wid = lax.axis_index("s") * _NC + lax.axis_index("c")
    sems = (sem0, sem1)
    pltpu.sync_copy(lse_hbm, lse_v)
    acc_v[...] = jnp.zeros((_L,), jnp.float32)
    c_lanes = [jnp.full((_L,), c, jnp.int32) for c in range(_CB)]
    zero16 = jnp.zeros((_L,), jnp.float32)

    def cblock_body(g):
        c0 = g * _CB
        pltpu.sync_copy(tabt_hbm.at[pl.ds(c0, _CB)], tbl_v)

        for ic in range(_NIC):
            p = ic % 2
            i0 = ic * _ICHUNK
            pltpu.sync_copy(idx_hbm.at[pl.ds(i0, _ICHUNK)], idx_v)
            pltpu.sync_copy(tgt_hbm.at[pl.ds(i0, _ICHUNK)], tgt_v)
            if ic >= 2:
                pltpu.make_async_copy(
                    out_v.at[p],
                    out_hbm.at[pl.ds(c0, _CB),
                               pl.ds((ic - 2) * _ICHUNK, _ICHUNK)],
                    sems[p]).wait()

            def igroup_body(v2, acc, p=p):
                for u in range(_UNROLL):
                    o = v2 * (_L * _UNROLL) + u * _L
                    idx16 = idx_v[pl.ds(o, _L)]
                    tgt16 = tgt_v[pl.ds(o, _L)]
                    for c in range(_CB):
                        row = plsc.load_gather(tbl_v, [c_lanes[c], idx16])
                        out_v[p, c, pl.ds(o, _L)] = row
                    tloc = tgt16 - c0
                    inblk = (tloc >= 0) & (tloc < _CB)
                    jc = jnp.clip(tloc, 0, _CB - 1)
                    tval = plsc.load_gather(tbl_v, [jc, idx16])
                    lse_g = plsc.load_gather(lse_v, [idx16])
                    acc = acc + jnp.where(inblk, lse_g - tval, zero16)
                return acc

            acc = lax.fori_loop(0, _ICHUNK // (_L * _UNROLL), igroup_body,
                                acc_v[...])
            acc_v[...] = acc
            pltpu.async_copy(
                out_v.at[p],
                out_hbm.at[pl.ds(c0, _CB), pl.ds(i0, _ICHUNK)],
                sems[p])

        for ic in (_NIC - 2, _NIC - 1):
            pltpu.make_async_copy(
                out_v.at[ic % 2],
                out_hbm.at[pl.ds(c0, _CB), pl.ds(ic * _ICHUNK, _ICHUNK)],
                sems[ic % 2]).wait()
        return g + _NW

    lax.while_loop(lambda g: g < _NCB, cblock_body, wid)
    pltpu.sync_copy(acc_v, part_hbm.at[wid])


def _loss_body(p_ref, out_ref):
    out_ref[...] = (jnp.sum(p_ref[...]) / N)[None]


def _loss_reduce(partials):
    return pl.pallas_call(
        _loss_body,
        out_shape=jax.ShapeDtypeStruct((1,), jnp.float32),
    )(partials)


def kernel(table, idx, targets):
    tableT, lse = _lse_and_transpose(table)
    idx_f = idx.reshape(-1)
    tgt_f = targets.reshape(-1)
    logitsT, partials = _sc_gather_t(tableT, idx_f, tgt_f, lse)
    loss = _loss_reduce(partials)[0]
    return logitsT.T, loss


# speedup vs baseline: 1.7289x; 1.5537x over previous
import functools

import jax
import jax.numpy as jnp
from jax import lax
from jax.experimental import pallas as pl
from jax.experimental.pallas import tpu as pltpu
from jax.experimental.pallas import tpu_sc as plsc

C = 1000
N = 1024 * 50

_info = plsc.get_sparse_core_info()
_NC, _NS, _L = _info.num_cores, _info.num_subcores, _info.num_lanes
_NW = _NC * _NS
_CB = 8
_NCB = C // _CB
_ICHUNK = 2560
_NIC = N // _ICHUNK
_UNROLL = 8


def _lse_t_body(tab_ref, tabt_ref, lse_ref):
    x = tab_ref[...]
    m = jnp.max(x, axis=1)
    s = jnp.sum(jnp.exp(x - m[:, None]), axis=1)
    lse_ref[...] = m + jnp.log(s)
    tabt_ref[...] = x.T


def _lse_and_transpose(table):
    return pl.pallas_call(
        _lse_t_body,
        out_shape=[
            jax.ShapeDtypeStruct((C, C), jnp.float32),
            jax.ShapeDtypeStruct((C,), jnp.float32),
        ],
    )(table)


_sc_mesh = plsc.VectorSubcoreMesh(core_axis_name="c", subcore_axis_name="s")


@functools.partial(
    pl.kernel,
    mesh=_sc_mesh,
    out_type=[
        jax.ShapeDtypeStruct((C, N), jnp.float32),
        jax.ShapeDtypeStruct((_NW, _L), jnp.float32),
    ],
    scratch_types=[
        pltpu.VMEM((_CB, C), jnp.float32),
        pltpu.VMEM((_ICHUNK,), jnp.int32),
        pltpu.VMEM((_ICHUNK,), jnp.int32),
        pltpu.VMEM((2, _CB, _ICHUNK), jnp.float32),
        pltpu.VMEM((C,), jnp.float32),
        pltpu.VMEM((_L,), jnp.float32),
        pltpu.SemaphoreType.DMA,
        pltpu.SemaphoreType.DMA,
    ],
    compiler_params=pltpu.CompilerParams(needs_layout_passes=False),
)
def _sc_gather_t(tabt_hbm, idx_hbm, tgt_hbm, lse_hbm, out_hbm, part_hbm,
                 tbl_v, idx_v, tgt_v, out_v, lse_v, acc_v, sem0, sem1):
    wid = lax.axis_index("s") * _NC + lax.axis_index("c")
    sems = (sem0, sem1)
    pltpu.sync_copy(lse_hbm, lse_v)
    acc_v[...] = jnp.zeros((_L,), jnp.float32)
    c_lanes = [jnp.full((_L,), c, jnp.int32) for c in range(_CB)]
    zero16 = jnp.zeros((_L,), jnp.float32)

    def cblock_body(g):
        c0 = g * _CB
        pltpu.sync_copy(tabt_hbm.at[pl.ds(c0, _CB)], tbl_v)

        for ic in range(_NIC):
            p = ic % 2
            i0 = ic * _ICHUNK
            pltpu.sync_copy(idx_hbm.at[pl.ds(i0, _ICHUNK)], idx_v)
            pltpu.sync_copy(tgt_hbm.at[pl.ds(i0, _ICHUNK)], tgt_v)
            if ic >= 2:
                pltpu.make_async_copy(
                    out_v.at[p],
                    out_hbm.at[pl.ds(c0, _CB),
                               pl.ds((ic - 2) * _ICHUNK, _ICHUNK)],
                    sems[p]).wait()

            def igroup_body(v2, acc, p=p):
                for u in range(_UNROLL):
                    o = v2 * (_L * _UNROLL) + u * _L
                    idx16 = idx_v[pl.ds(o, _L)]
                    tgt16 = tgt_v[pl.ds(o, _L)]
                    rows = [plsc.load_gather(tbl_v, [c_lanes[c], idx16])
                            for c in range(_CB)]
                    for c in range(_CB):
                        out_v[p, c, pl.ds(o, _L)] = rows[c]
                    tloc = tgt16 - c0
                    inblk = (tloc >= 0) & (tloc < _CB)
                    jc = jnp.clip(tloc, 0, _CB - 1)
                    tval = plsc.load_gather(tbl_v, [jc, idx16])
                    lse_g = plsc.load_gather(lse_v, [idx16])
                    acc = acc + jnp.where(inblk, lse_g - tval, zero16)
                return acc

            acc = lax.fori_loop(0, _ICHUNK // (_L * _UNROLL), igroup_body,
                                acc_v[...])
            acc_v[...] = acc
            pltpu.async_copy(
                out_v.at[p],
                out_hbm.at[pl.ds(c0, _CB), pl.ds(i0, _ICHUNK)],
                sems[p])

        for ic in (_NIC - 2, _NIC - 1):
            pltpu.make_async_copy(
                out_v.at[ic % 2],
                out_hbm.at[pl.ds(c0, _CB), pl.ds(ic * _ICHUNK, _ICHUNK)],
                sems[ic % 2]).wait()
        return g + _NW

    lax.while_loop(lambda g: g < _NCB, cblock_body, wid)
    pltpu.sync_copy(acc_v, part_hbm.at[wid])


def _loss_body(p_ref, out_ref):
    out_ref[...] = (jnp.sum(p_ref[...]) / N)[None]


def _loss_reduce(partials):
    return pl.pallas_call(
        _loss_body,
        out_shape=jax.ShapeDtypeStruct((1,), jnp.float32),
    )(partials)


def kernel(table, idx, targets):
    tableT, lse = _lse_and_transpose(table)
    idx_f = idx.reshape(-1)
    tgt_f = targets.reshape(-1)
    logitsT, partials = _sc_gather_t(tableT, idx_f, tgt_f, lse)
    loss = _loss_reduce(partials)[0]
    return logitsT.T, loss


# speedup vs baseline: 2.1153x; 1.2235x over previous
import functools

import jax
import jax.numpy as jnp
from jax import lax
from jax.experimental import pallas as pl
from jax.experimental.pallas import tpu as pltpu
from jax.experimental.pallas import tpu_sc as plsc

C = 1000
N = 1024 * 50

_info = plsc.get_sparse_core_info()
_NC, _NS, _L = _info.num_cores, _info.num_subcores, _info.num_lanes
_NW = _NC * _NS
_CB = 8
_NCB = C // _CB
_ICHUNK = 2560
_NIC = N // _ICHUNK
_UNROLL = 8
_PER_W = N // _NW
_LCH = 128


def _lse_t_body(tab_ref, tabt_ref, lse_ref):
    x = tab_ref[...]
    m = jnp.max(x, axis=1)
    s = jnp.sum(jnp.exp(x - m[:, None]), axis=1)
    lse_ref[...] = m + jnp.log(s)
    tabt_ref[...] = x.T


def _lse_and_transpose(table):
    return pl.pallas_call(
        _lse_t_body,
        out_shape=[
            jax.ShapeDtypeStruct((C, C), jnp.float32),
            jax.ShapeDtypeStruct((C,), jnp.float32),
        ],
    )(table)


_sc_mesh = plsc.VectorSubcoreMesh(core_axis_name="c", subcore_axis_name="s")


@functools.partial(
    pl.kernel,
    mesh=_sc_mesh,
    out_type=[
        jax.ShapeDtypeStruct((C, N), jnp.float32),
        jax.ShapeDtypeStruct((_NW, _L), jnp.float32),
    ],
    scratch_types=[
        pltpu.VMEM((_CB, C), jnp.float32),
        pltpu.VMEM((_ICHUNK,), jnp.int32),
        pltpu.VMEM((_LCH,), jnp.int32),
        pltpu.VMEM((_LCH,), jnp.int32),
        pltpu.VMEM((_LCH,), jnp.float32),
        pltpu.VMEM((2, _CB, _ICHUNK), jnp.float32),
        pltpu.VMEM((C,), jnp.float32),
        pltpu.VMEM((_L,), jnp.float32),
        pltpu.SemaphoreType.DMA,
        pltpu.SemaphoreType.DMA,
    ],
    compiler_params=pltpu.CompilerParams(needs_layout_passes=False),
)
def _sc_gather_t(tabt_hbm, tabflat_hbm, idx_hbm, tgt_hbm, lse_hbm,
                 out_hbm, part_hbm,
                 tbl_v, idx_v, tgt_v, fidx_v, tval_v, out_v, lse_v, acc_v,
                 sem0, sem1):
    wid = lax.axis_index("s") * _NC + lax.axis_index("c")
    sems = (sem0, sem1)
    pltpu.sync_copy(lse_hbm, lse_v)
    c_lanes = [jnp.full((_L,), c, jnp.int32) for c in range(_CB)]

    acc = jnp.zeros((_L,), jnp.float32)
    base = wid * _PER_W
    for lc in range(_PER_W // _LCH):
        o0 = base + lc * _LCH
        pltpu.sync_copy(idx_hbm.at[pl.ds(o0, _LCH)], idx_v.at[pl.ds(0, _LCH)])
        pltpu.sync_copy(tgt_hbm.at[pl.ds(o0, _LCH)], tgt_v)
        for v in range(_LCH // _L):
            idx16 = idx_v[pl.ds(v * _L, _L)]
            tgt16 = tgt_v[pl.ds(v * _L, _L)]
            fidx_v[pl.ds(v * _L, _L)] = idx16 * C + tgt16
        pltpu.async_copy(tabflat_hbm.at[fidx_v], tval_v, sem0).wait()
        for v in range(_LCH // _L):
            idx16 = idx_v[pl.ds(v * _L, _L)]
            lse_g = plsc.load_gather(lse_v, [idx16])
            acc = acc + (lse_g - tval_v[pl.ds(v * _L, _L)])
    acc_v[...] = acc
    pltpu.sync_copy(acc_v, part_hbm.at[wid])

    def cblock_body(g):
        c0 = g * _CB
        pltpu.sync_copy(tabt_hbm.at[pl.ds(c0, _CB)], tbl_v)

        for ic in range(_NIC):
            p = ic % 2
            i0 = ic * _ICHUNK
            pltpu.sync_copy(idx_hbm.at[pl.ds(i0, _ICHUNK)], idx_v)
            if ic >= 2:
                pltpu.make_async_copy(
                    out_v.at[p],
                    out_hbm.at[pl.ds(c0, _CB),
                               pl.ds((ic - 2) * _ICHUNK, _ICHUNK)],
                    sems[p]).wait()

            def igroup_body(v2, carry, p=p):
                prev_rows = None
                prev_o = 0
                for u in range(_UNROLL):
                    o = v2 * (_L * _UNROLL) + u * _L
                    idx16 = idx_v[pl.ds(o, _L)]
                    rows = [plsc.load_gather(tbl_v, [c_lanes[c], idx16])
                            for c in range(_CB)]
                    if prev_rows is not None:
                        for c in range(_CB):
                            out_v[p, c, pl.ds(prev_o, _L)] = prev_rows[c]
                    prev_rows, prev_o = rows, o
                for c in range(_CB):
                    out_v[p, c, pl.ds(prev_o, _L)] = prev_rows[c]
                return carry

            lax.fori_loop(0, _ICHUNK // (_L * _UNROLL), igroup_body, 0)
            pltpu.async_copy(
                out_v.at[p],
                out_hbm.at[pl.ds(c0, _CB), pl.ds(i0, _ICHUNK)],
                sems[p])

        for ic in (_NIC - 2, _NIC - 1):
            pltpu.make_async_copy(
                out_v.at[ic % 2],
                out_hbm.at[pl.ds(c0, _CB), pl.ds(ic * _ICHUNK, _ICHUNK)],
                sems[ic % 2]).wait()
        return g + _NW

    lax.while_loop(lambda g: g < _NCB, cblock_body, wid)


def _loss_body(p_ref, out_ref):
    out_ref[...] = (jnp.sum(p_ref[...]) / N)[None]


def _loss_reduce(partials):
    return pl.pallas_call(
        _loss_body,
        out_shape=jax.ShapeDtypeStruct((1,), jnp.float32),
    )(partials)


def kernel(table, idx, targets):
    tableT, lse = _lse_and_transpose(table)
    idx_f = idx.reshape(-1)
    tgt_f = targets.reshape(-1)
    logitsT, partials = _sc_gather_t(tableT, table.reshape(-1),
                                     idx_f, tgt_f, lse)
    loss = _loss_reduce(partials)[0]
    return logitsT.T, loss
